# baseline (speedup 1.0000x reference)
import functools

import jax
import jax.numpy as jnp
from jax import lax
from jax.experimental import pallas as pl
from jax.experimental.pallas import tpu as pltpu
from jax.experimental.pallas import tpu_sc as plsc

N = 50000
E = 1600000
F_IN = 128
H = 32
C = 2
R = 2000


def _phase_a_body(x_ref, w_ref, b_ref, o_ref):
    o_ref[...] = (
        jnp.dot(x_ref[...], w_ref[...], preferred_element_type=jnp.float32)
        + b_ref[...]
    )


def _phase_a(x, w_t, b):
    grid = N // R
    return pl.pallas_call(
        _phase_a_body,
        grid=(grid,),
        in_specs=[
            pl.BlockSpec((R, F_IN), lambda i: (i, 0)),
            pl.BlockSpec((F_IN, H), lambda i: (0, 0)),
            pl.BlockSpec((1, H), lambda i: (0, 0)),
        ],
        out_specs=pl.BlockSpec((R, H), lambda i: (i, 0)),
        out_shape=jax.ShapeDtypeStruct((N, H), jnp.float32),
    )(x, w_t, b)


def _gru_from_parts(x1, x2, h, p):
    (a1r, a1z, a1n, a2r, a2z, a2n, bgr, bgz, bgn,
     whr, whz, whn, bhr, bhz, bhn) = p
    dot = lambda a, b: jnp.dot(a, b, preferred_element_type=jnp.float32)
    gir = dot(x1, a1r) + dot(x2, a2r) + bgr
    giz = dot(x1, a1z) + dot(x2, a2z) + bgz
    gin = dot(x1, a1n) + dot(x2, a2n) + bgn
    ghr = dot(h, whr) + bhr
    ghz = dot(h, whz) + bhz
    ghn = dot(h, whn) + bhn
    r = jax.nn.sigmoid(gir + ghr)
    z = jax.nn.sigmoid(giz + ghz)
    n = jnp.tanh(gin + r * ghn)
    return (1.0 - z) * n + z * h


def _phase_b_body(x1_ref, x2_ref, h_ref, *rest):
    (a1r, a1z, a1n, a2r, a2z, a2n, bgr, bgz, bgn,
     whr, whz, whn, bhr, bhz, bhn, wlin, blin, o_ref) = rest
    hn = _gru_from_parts(
        x1_ref[...], x2_ref[...], h_ref[...],
        (a1r[...], a1z[...], a1n[...], a2r[...], a2z[...], a2n[...],
         bgr[...], bgz[...], bgn[...], whr[...], whz[...], whn[...],
         bhr[...], bhz[...], bhn[...]))
    o_ref[...] = jnp.dot(hn, wlin[...], preferred_element_type=jnp.float32) + blin[...]


def _phase_b(x1, x2, h, parts, wlin_t, blin):
    grid = N // R
    mat = pl.BlockSpec((H, H), lambda i: (0, 0))
    vec = pl.BlockSpec((1, H), lambda i: (0, 0))
    row = pl.BlockSpec((R, H), lambda i: (i, 0))
    return pl.pallas_call(
        _phase_b_body,
        grid=(grid,),
        in_specs=[row, row, row] + [mat] * 6 + [vec] * 3 + [mat] * 3 + [vec] * 3
        + [mat, vec],
        out_specs=row,
        out_shape=jax.ShapeDtypeStruct((N, H), jnp.float32),
    )(x1, x2, h, *parts, wlin_t, blin)


def _phase_c_body(x1_ref, x2_ref, h_ref, *rest):
    (a1r, a1z, a1n, a2r, a2z, a2n, bgr, bgz, bgn,
     whr, whz, whn, bhr, bhz, bhn, wout, bout, o_ref) = rest
    hn = _gru_from_parts(
        x1_ref[...], x2_ref[...], h_ref[...],
        (a1r[...], a1z[...], a1n[...], a2r[...], a2z[...], a2n[...],
         bgr[...], bgz[...], bgn[...], whr[...], whz[...], whn[...],
         bhr[...], bhz[...], bhn[...]))
    logits = jnp.dot(hn, wout[...], preferred_element_type=jnp.float32) + bout[...]
    m = jnp.max(logits, axis=-1, keepdims=True)
    s = jnp.sum(jnp.exp(logits - m), axis=-1, keepdims=True)
    o_ref[...] = logits - m - jnp.log(s)


def _phase_c(x1, x2, h, parts, wout_t, bout):
    grid = N // R
    mat = pl.BlockSpec((H, H), lambda i: (0, 0))
    vec = pl.BlockSpec((1, H), lambda i: (0, 0))
    row = pl.BlockSpec((R, H), lambda i: (i, 0))
    return pl.pallas_call(
        _phase_c_body,
        grid=(grid,),
        in_specs=[row, row, row] + [mat] * 6 + [vec] * 3 + [mat] * 3 + [vec] * 3
        + [pl.BlockSpec((H, C), lambda i: (0, 0)), pl.BlockSpec((1, C), lambda i: (0, 0))],
        out_specs=pl.BlockSpec((R, C), lambda i: (i, 0)),
        out_shape=jax.ShapeDtypeStruct((N, C), jnp.float32),
    )(x1, x2, h, *parts, wout_t, bout)


def _fold_gru_parts(W_con, b_con, W_ih, b_ih, W_hh, b_hh):
    a1 = W_con[:, :H].T @ W_ih.T
    a2 = W_con[:, H:].T @ W_ih.T
    bg = (b_con @ W_ih.T + b_ih)[None]
    wh = W_hh.T
    bh = b_hh[None]
    sl = lambda m, k: m[:, k * H:(k + 1) * H]
    vl = lambda v, k: v[:, k * H:(k + 1) * H]
    return (sl(a1, 0), sl(a1, 1), sl(a1, 2),
            sl(a2, 0), sl(a2, 1), sl(a2, 2),
            vl(bg, 0), vl(bg, 1), vl(bg, 2),
            sl(wh, 0), sl(wh, 1), sl(wh, 2),
            vl(bh, 0), vl(bh, 1), vl(bh, 2))


SUB = 128
CHUNK_ROWS = 2
CHUNK_E = SUB * CHUNK_ROWS
N_TILES = 16
EPAD = 1638400
ROWS2D = EPAD // SUB
EDGES_PER_TILE = EPAD // N_TILES
ROWS_PER_TILE = EDGES_PER_TILE // SUB
N_CHUNKS = ROWS_PER_TILE // CHUNK_ROWS
NPAD = 50048
NODES_PER_TILE = NPAD // N_TILES


def _conv_pair(y, src2d, dst2d, w_all):
    mesh = plsc.VectorSubcoreMesh(core_axis_name="c", subcore_axis_name="s")

    @functools.partial(
        pl.kernel,
        mesh=mesh,
        compiler_params=pltpu.CompilerParams(use_tc_tiling_on_sc=False),
        out_type=jax.ShapeDtypeStruct((2, NPAD, H), jnp.float32),
        scratch_types=[
            pltpu.VMEM((CHUNK_ROWS, SUB), jnp.int32),
            pltpu.VMEM((CHUNK_ROWS, SUB), jnp.int32),
            pltpu.VMEM((CHUNK_ROWS, SUB), jnp.int32),
            pltpu.VMEM((CHUNK_ROWS, SUB), jnp.int32),
            pltpu.VMEM((CHUNK_ROWS, SUB), jnp.float32),
            pltpu.VMEM((CHUNK_ROWS, SUB), jnp.float32),
            pltpu.VMEM((CHUNK_E, H), jnp.float32),
            pltpu.VMEM((CHUNK_E, H), jnp.float32),
            pltpu.VMEM_SHARED((NPAD, H), jnp.float32),
            pltpu.SemaphoreType.DMA,
            pltpu.SemaphoreType.DMA,
        ],
    )
    def k(y_hbm, src_hbm, dst_hbm, w_hbm, out_hbm,
          sidx0, sidx1, didx0, didx1, wv0, wv1, rows0, rows1, acc,
          sem0, sem1):
        c = lax.axis_index("c")
        s = lax.axis_index("s")
        sidx = (sidx0, sidx1)
        didx = (didx0, didx1)
        w_v = (wv0, wv1)
        rows = (rows0, rows1)
        sem = (sem0, sem1)

        z = jnp.zeros((16,), jnp.float32)

        def zbody(r, carry):
            rows0[r, pl.ds(0, 16)] = z
            rows0[r, pl.ds(16, 16)] = z
            return carry

        lax.fori_loop(0, CHUNK_E, zbody, 0)
        base = s * NODES_PER_TILE
        for q in range(NODES_PER_TILE // CHUNK_E):
            pltpu.sync_copy(rows0, acc.at[pl.ds(base + q * CHUNK_E, CHUNK_E)])
        rem = NODES_PER_TILE % CHUNK_E
        if rem:
            pltpu.sync_copy(rows0.at[pl.ds(0, rem)],
                            acc.at[pl.ds(base + NODES_PER_TILE - rem, rem)])
        plsc.subcore_barrier()

        def load_idx(g, b):
            r0 = pl.multiple_of(s * ROWS_PER_TILE + g * CHUNK_ROWS, CHUNK_ROWS)
            pltpu.sync_copy(src_hbm.at[c, pl.ds(r0, CHUNK_ROWS)], sidx[b])
            pltpu.sync_copy(dst_hbm.at[c, pl.ds(r0, CHUNK_ROWS)], didx[b])
            pltpu.sync_copy(w_hbm.at[c, pl.ds(r0, CHUNK_ROWS)], w_v[b])

        def fire_gathers(b):
            for j in range(CHUNK_ROWS):
                pltpu.async_copy(y_hbm.at[sidx[b].at[j]],
                                 rows[b].at[pl.ds(j * SUB, SUB)], sem[b])

        def drain_gathers(b):
            pltpu.make_async_copy(y_hbm.at[pl.ds(0, CHUNK_E)], rows[b],
                                  sem[b]).wait()

        load_idx(0, 0)
        fire_gathers(0)

        def pair_body(pg, carry):
            for b in range(2):
                g = 2 * pg + b
                b1 = 1 - b

                @pl.when(g < N_CHUNKS - 1)
                def _prefetch():
                    load_idx(g + 1, b1)
                    fire_gathers(b1)

                drain_gathers(b)
                rb = rows[b]
                wb = w_v[b]

                def scale(t, cc):
                    j = t >> 3
                    w16 = wb[j, pl.ds((t & 7) * 16, 16)]
                    for l in range(16):
                        lv = jnp.full((16,), l, jnp.int32)
                        ws = w16.at[lv].get(mode="promise_in_bounds")
                        e = t * 16 + l
                        rb[e, pl.ds(0, 16)] = rb[e, pl.ds(0, 16)] * ws
                        rb[e, pl.ds(16, 16)] = rb[e, pl.ds(16, 16)] * ws
                    return cc

                lax.fori_loop(0, CHUNK_E // 16, scale, 0)
                for j in range(CHUNK_ROWS):
                    pltpu.sync_copy(rb.at[pl.ds(j * SUB, SUB)],
                                    acc.at[didx[b].at[j]], add=True)
            return carry

        lax.fori_loop(0, N_CHUNKS // 2, pair_body, 0)
        plsc.subcore_barrier()
        pltpu.sync_copy(acc.at[pl.ds(base, NODES_PER_TILE)],
                        out_hbm.at[c, pl.ds(base, NODES_PER_TILE)])

    out = k(y, src2d, dst2d, w_all)
    return out[0, :N], out[1, :N]


def kernel(x, edge_index, edge_weight, edge_index_re, edge_weight_re,
           W_first, b_first, W_con1, b_con1, W_con2, b_con2,
           W_lin1, b_lin1, W_out, b_out, W_ih, W_hh, b_ih, b_hh):
    parts1 = _fold_gru_parts(W_con1, b_con1, W_ih, b_ih, W_hh, b_hh)
    parts2 = _fold_gru_parts(W_con2, b_con2, W_ih, b_ih, W_hh, b_hh)

    pad_i = jnp.zeros((EPAD - E,), jnp.int32)
    pad_f = jnp.zeros((EPAD - E,), jnp.float32)
    cat_i = lambda a: jnp.concatenate([a, pad_i]).reshape(ROWS2D, SUB)
    cat_f = lambda a: jnp.concatenate([a, pad_f]).reshape(ROWS2D, SUB)
    src2d = jnp.stack([cat_i(edge_index[0]), cat_i(edge_index_re[0])])
    dst2d = jnp.stack([cat_i(edge_index[1]), cat_i(edge_index_re[1])])
    w_all = jnp.stack([cat_f(edge_weight), cat_f(edge_weight_re)])

    y = _phase_a(x, W_first.T, b_first[None])
    x1, x2 = _conv_pair(y, src2d, dst2d, w_all)
    xm = _phase_b(x1, x2, y, parts1, W_lin1.T, b_lin1[None])
    x1b, x2b = _conv_pair(xm, src2d, dst2d, w_all)
    return _phase_c(x1b, x2b, y, parts2, W_out.T, b_out[None])


# speedup vs baseline: 1.2564x; 1.2564x over previous
import functools

import jax
import jax.numpy as jnp
from jax import lax
from jax.experimental import pallas as pl
from jax.experimental.pallas import tpu as pltpu
from jax.experimental.pallas import tpu_sc as plsc

N = 50000
E = 1600000
F_IN = 128
H = 32
C = 2
R = 2000


def _phase_a_body(x_ref, w_ref, b_ref, o_ref):
    o_ref[...] = (
        jnp.dot(x_ref[...], w_ref[...], preferred_element_type=jnp.float32)
        + b_ref[...]
    )


def _phase_a(x, w_t, b):
    grid = N // R
    return pl.pallas_call(
        _phase_a_body,
        grid=(grid,),
        in_specs=[
            pl.BlockSpec((R, F_IN), lambda i: (i, 0)),
            pl.BlockSpec((F_IN, H), lambda i: (0, 0)),
            pl.BlockSpec((1, H), lambda i: (0, 0)),
        ],
        out_specs=pl.BlockSpec((R, H), lambda i: (i, 0)),
        out_shape=jax.ShapeDtypeStruct((N, H), jnp.float32),
    )(x, w_t, b)


def _gru_from_parts(x1, x2, h, p):
    (a1r, a1z, a1n, a2r, a2z, a2n, bgr, bgz, bgn,
     whr, whz, whn, bhr, bhz, bhn) = p
    dot = lambda a, b: jnp.dot(a, b, preferred_element_type=jnp.float32)
    gir = dot(x1, a1r) + dot(x2, a2r) + bgr
    giz = dot(x1, a1z) + dot(x2, a2z) + bgz
    gin = dot(x1, a1n) + dot(x2, a2n) + bgn
    ghr = dot(h, whr) + bhr
    ghz = dot(h, whz) + bhz
    ghn = dot(h, whn) + bhn
    r = jax.nn.sigmoid(gir + ghr)
    z = jax.nn.sigmoid(giz + ghz)
    n = jnp.tanh(gin + r * ghn)
    return (1.0 - z) * n + z * h


def _phase_b_body(x1_ref, x2_ref, h_ref, *rest):
    (a1r, a1z, a1n, a2r, a2z, a2n, bgr, bgz, bgn,
     whr, whz, whn, bhr, bhz, bhn, wlin, blin, o_ref) = rest
    hn = _gru_from_parts(
        x1_ref[...], x2_ref[...], h_ref[...],
        (a1r[...], a1z[...], a1n[...], a2r[...], a2z[...], a2n[...],
         bgr[...], bgz[...], bgn[...], whr[...], whz[...], whn[...],
         bhr[...], bhz[...], bhn[...]))
    o_ref[...] = jnp.dot(hn, wlin[...], preferred_element_type=jnp.float32) + blin[...]


def _phase_b(x1, x2, h, parts, wlin_t, blin):
    grid = N // R
    mat = pl.BlockSpec((H, H), lambda i: (0, 0))
    vec = pl.BlockSpec((1, H), lambda i: (0, 0))
    row = pl.BlockSpec((R, H), lambda i: (i, 0))
    return pl.pallas_call(
        _phase_b_body,
        grid=(grid,),
        in_specs=[row, row, row] + [mat] * 6 + [vec] * 3 + [mat] * 3 + [vec] * 3
        + [mat, vec],
        out_specs=row,
        out_shape=jax.ShapeDtypeStruct((N, H), jnp.float32),
    )(x1, x2, h, *parts, wlin_t, blin)


def _phase_c_body(x1_ref, x2_ref, h_ref, *rest):
    (a1r, a1z, a1n, a2r, a2z, a2n, bgr, bgz, bgn,
     whr, whz, whn, bhr, bhz, bhn, wout, bout, o_ref) = rest
    hn = _gru_from_parts(
        x1_ref[...], x2_ref[...], h_ref[...],
        (a1r[...], a1z[...], a1n[...], a2r[...], a2z[...], a2n[...],
         bgr[...], bgz[...], bgn[...], whr[...], whz[...], whn[...],
         bhr[...], bhz[...], bhn[...]))
    logits = jnp.dot(hn, wout[...], preferred_element_type=jnp.float32) + bout[...]
    m = jnp.max(logits, axis=-1, keepdims=True)
    s = jnp.sum(jnp.exp(logits - m), axis=-1, keepdims=True)
    o_ref[...] = logits - m - jnp.log(s)


def _phase_c(x1, x2, h, parts, wout_t, bout):
    grid = N // R
    mat = pl.BlockSpec((H, H), lambda i: (0, 0))
    vec = pl.BlockSpec((1, H), lambda i: (0, 0))
    row = pl.BlockSpec((R, H), lambda i: (i, 0))
    return pl.pallas_call(
        _phase_c_body,
        grid=(grid,),
        in_specs=[row, row, row] + [mat] * 6 + [vec] * 3 + [mat] * 3 + [vec] * 3
        + [pl.BlockSpec((H, C), lambda i: (0, 0)), pl.BlockSpec((1, C), lambda i: (0, 0))],
        out_specs=pl.BlockSpec((R, C), lambda i: (i, 0)),
        out_shape=jax.ShapeDtypeStruct((N, C), jnp.float32),
    )(x1, x2, h, *parts, wout_t, bout)


def _fold_gru_parts(W_con, b_con, W_ih, b_ih, W_hh, b_hh):
    a1 = W_con[:, :H].T @ W_ih.T
    a2 = W_con[:, H:].T @ W_ih.T
    bg = (b_con @ W_ih.T + b_ih)[None]
    wh = W_hh.T
    bh = b_hh[None]
    sl = lambda m, k: m[:, k * H:(k + 1) * H]
    vl = lambda v, k: v[:, k * H:(k + 1) * H]
    return (sl(a1, 0), sl(a1, 1), sl(a1, 2),
            sl(a2, 0), sl(a2, 1), sl(a2, 2),
            vl(bg, 0), vl(bg, 1), vl(bg, 2),
            sl(wh, 0), sl(wh, 1), sl(wh, 2),
            vl(bh, 0), vl(bh, 1), vl(bh, 2))


SUB = 128
CHUNK_ROWS = 2
CHUNK_E = SUB * CHUNK_ROWS
BLK_CHUNKS = 8
BLK_ROWS = BLK_CHUNKS * CHUNK_ROWS
N_TILES = 16
EPAD = 1638400
ROWS2D = EPAD // SUB
EDGES_PER_TILE = EPAD // N_TILES
ROWS_PER_TILE = EDGES_PER_TILE // SUB
N_CHUNKS = ROWS_PER_TILE // CHUNK_ROWS
N_BLKS = N_CHUNKS // BLK_CHUNKS
NPAD = 50048
NODES_PER_TILE = NPAD // N_TILES


def _conv_pair(y, src2d, dst2d, w_all):
    mesh = plsc.VectorSubcoreMesh(core_axis_name="c", subcore_axis_name="s")

    @functools.partial(
        pl.kernel,
        mesh=mesh,
        compiler_params=pltpu.CompilerParams(use_tc_tiling_on_sc=False),
        out_type=jax.ShapeDtypeStruct((2, NPAD, H), jnp.float32),
        scratch_types=[
            pltpu.VMEM((BLK_ROWS, SUB), jnp.int32),
            pltpu.VMEM((BLK_ROWS, SUB), jnp.int32),
            pltpu.VMEM((BLK_ROWS, SUB), jnp.int32),
            pltpu.VMEM((BLK_ROWS, SUB), jnp.int32),
            pltpu.VMEM((BLK_ROWS, SUB), jnp.float32),
            pltpu.VMEM((BLK_ROWS, SUB), jnp.float32),
            pltpu.VMEM((CHUNK_E, H), jnp.float32),
            pltpu.VMEM((CHUNK_E, H), jnp.float32),
            pltpu.VMEM_SHARED((NPAD, H), jnp.float32),
            pltpu.SemaphoreType.DMA,
            pltpu.SemaphoreType.DMA,
            pltpu.SemaphoreType.DMA,
            pltpu.SemaphoreType.DMA,
            pltpu.SemaphoreType.DMA,
            pltpu.SemaphoreType.DMA,
        ],
    )
    def k(y_hbm, src_hbm, dst_hbm, w_hbm, out_hbm,
          sidx0, sidx1, didx0, didx1, wv0, wv1, rows0, rows1, acc,
          semg0, semg1, sems0, sems1, semi0, semi1):
        c = lax.axis_index("c")
        s = lax.axis_index("s")
        sidx = (sidx0, sidx1)
        didx = (didx0, didx1)
        w_v = (wv0, wv1)
        rows = (rows0, rows1)
        sem_g = (semg0, semg1)
        sem_s = (sems0, sems1)
        sem_i = (semi0, semi1)

        z = jnp.zeros((16,), jnp.float32)

        def zbody(r, carry):
            rows0[r, pl.ds(0, 16)] = z
            rows0[r, pl.ds(16, 16)] = z
            return carry

        lax.fori_loop(0, CHUNK_E, zbody, 0)
        base = s * NODES_PER_TILE
        for q in range(NODES_PER_TILE // CHUNK_E):
            pltpu.sync_copy(rows0, acc.at[pl.ds(base + q * CHUNK_E, CHUNK_E)])
        rem = NODES_PER_TILE % CHUNK_E
        if rem:
            pltpu.sync_copy(rows0.at[pl.ds(0, rem)],
                            acc.at[pl.ds(base + NODES_PER_TILE - rem, rem)])
        plsc.subcore_barrier()

        def issue_idx_block(blk, q):
            r0 = pl.multiple_of(s * ROWS_PER_TILE + blk * BLK_ROWS, BLK_ROWS)
            pltpu.async_copy(src_hbm.at[c, pl.ds(r0, BLK_ROWS)], sidx[q], sem_i[q])
            pltpu.async_copy(dst_hbm.at[c, pl.ds(r0, BLK_ROWS)], didx[q], sem_i[q])
            pltpu.async_copy(w_hbm.at[c, pl.ds(r0, BLK_ROWS)], w_v[q], sem_i[q])

        def drain_idx_block(q):
            pltpu.make_async_copy(src_hbm.at[c, pl.ds(0, BLK_ROWS)], sidx[q],
                                  sem_i[q]).wait()
            pltpu.make_async_copy(dst_hbm.at[c, pl.ds(0, BLK_ROWS)], didx[q],
                                  sem_i[q]).wait()
            pltpu.make_async_copy(w_hbm.at[c, pl.ds(0, BLK_ROWS)], w_v[q],
                                  sem_i[q]).wait()

        def fire_gathers(q, row0, b):
            for j in range(CHUNK_ROWS):
                pltpu.async_copy(y_hbm.at[sidx[q].at[row0 + j]],
                                 rows[b].at[pl.ds(j * SUB, SUB)], sem_g[b])

        def drain_gathers(b):
            pltpu.make_async_copy(y_hbm.at[pl.ds(0, CHUNK_E)], rows[b],
                                  sem_g[b]).wait()

        def fire_scatters(q, row0, b):
            for j in range(CHUNK_ROWS):
                pltpu.async_copy(rows[b].at[pl.ds(j * SUB, SUB)],
                                 acc.at[didx[q].at[row0 + j]], sem_s[b],
                                 add=True)

        def drain_scatters(b):
            pltpu.make_async_copy(y_hbm.at[pl.ds(0, CHUNK_E)], rows[b],
                                  sem_s[b]).wait()

        def scale_chunk(b, q, crow):
            rb = rows[b]

            def scale(t, cc):
                w16 = w_v[q][crow + (t >> 3), pl.ds((t & 7) * 16, 16)]
                for l in range(16):
                    lv = jnp.full((16,), l, jnp.int32)
                    ws = w16.at[lv].get(mode="promise_in_bounds")
                    e = t * 16 + l
                    rb[e, pl.ds(0, 16)] = rb[e, pl.ds(0, 16)] * ws
                    rb[e, pl.ds(16, 16)] = rb[e, pl.ds(16, 16)] * ws
                return cc

            lax.fori_loop(0, CHUNK_E // 16, scale, 0)

        issue_idx_block(0, 0)
        drain_idx_block(0)
        fire_gathers(0, 0, 0)

        def blk_pair_body(pb, carry):
            for q in range(2):
                sb = 2 * pb + q
                q1 = 1 - q
                for cidx in range(BLK_CHUNKS):
                    b = cidx % 2
                    b1 = 1 - b

                    if cidx == 0:
                        @pl.when(sb > 0)
                        def _drain_prev():
                            drain_scatters(b1)

                        @pl.when(sb < N_BLKS - 1)
                        def _refill():
                            issue_idx_block(sb + 1, q1)
                    else:
                        drain_scatters(b1)

                    if cidx < BLK_CHUNKS - 1:
                        fire_gathers(q, (cidx + 1) * CHUNK_ROWS, b1)
                    else:
                        @pl.when(sb < N_BLKS - 1)
                        def _next_blk_gather():
                            drain_idx_block(q1)
                            fire_gathers(q1, 0, b1)

                    drain_gathers(b)
                    scale_chunk(b, q, cidx * CHUNK_ROWS)
                    fire_scatters(q, cidx * CHUNK_ROWS, b)
            return carry

        lax.fori_loop(0, N_BLKS // 2, blk_pair_body, 0)
        drain_scatters(1)
        plsc.subcore_barrier()
        pltpu.sync_copy(acc.at[pl.ds(base, NODES_PER_TILE)],
                        out_hbm.at[c, pl.ds(base, NODES_PER_TILE)])

    out = k(y, src2d, dst2d, w_all)
    return out[0, :N], out[1, :N]


def kernel(x, edge_index, edge_weight, edge_index_re, edge_weight_re,
           W_first, b_first, W_con1, b_con1, W_con2, b_con2,
           W_lin1, b_lin1, W_out, b_out, W_ih, W_hh, b_ih, b_hh):
    parts1 = _fold_gru_parts(W_con1, b_con1, W_ih, b_ih, W_hh, b_hh)
    parts2 = _fold_gru_parts(W_con2, b_con2, W_ih, b_ih, W_hh, b_hh)

    pad_i = jnp.zeros((EPAD - E,), jnp.int32)
    pad_f = jnp.zeros((EPAD - E,), jnp.float32)
    cat_i = lambda a: jnp.concatenate([a, pad_i]).reshape(ROWS2D, SUB)
    cat_f = lambda a: jnp.concatenate([a, pad_f]).reshape(ROWS2D, SUB)
    src2d = jnp.stack([cat_i(edge_index[0]), cat_i(edge_index_re[0])])
    dst2d = jnp.stack([cat_i(edge_index[1]), cat_i(edge_index_re[1])])
    w_all = jnp.stack([cat_f(edge_weight), cat_f(edge_weight_re)])

    y = _phase_a(x, W_first.T, b_first[None])
    x1, x2 = _conv_pair(y, src2d, dst2d, w_all)
    xm = _phase_b(x1, x2, y, parts1, W_lin1.T, b_lin1[None])
    x1b, x2b = _conv_pair(xm, src2d, dst2d, w_all)
    return _phase_c(x1b, x2b, y, parts2, W_out.T, b_out[None])


# speedup vs baseline: 1.2650x; 1.0068x over previous
import functools

import jax
import jax.numpy as jnp
from jax import lax
from jax.experimental import pallas as pl
from jax.experimental.pallas import tpu as pltpu
from jax.experimental.pallas import tpu_sc as plsc

N = 50000
E = 1600000
F_IN = 128
H = 32
C = 2
R = 2000


def _phase_a_body(x_ref, w_ref, b_ref, o_ref):
    o_ref[...] = (
        jnp.dot(x_ref[...], w_ref[...], preferred_element_type=jnp.float32)
        + b_ref[...]
    )


def _phase_a(x, w_t, b):
    grid = N // R
    return pl.pallas_call(
        _phase_a_body,
        grid=(grid,),
        in_specs=[
            pl.BlockSpec((R, F_IN), lambda i: (i, 0)),
            pl.BlockSpec((F_IN, H), lambda i: (0, 0)),
            pl.BlockSpec((1, H), lambda i: (0, 0)),
        ],
        out_specs=pl.BlockSpec((R, H), lambda i: (i, 0)),
        out_shape=jax.ShapeDtypeStruct((N, H), jnp.float32),
    )(x, w_t, b)


def _gru_from_parts(x1, x2, h, p):
    (a1r, a1z, a1n, a2r, a2z, a2n, bgr, bgz, bgn,
     whr, whz, whn, bhr, bhz, bhn) = p
    dot = lambda a, b: jnp.dot(a, b, preferred_element_type=jnp.float32)
    gir = dot(x1, a1r) + dot(x2, a2r) + bgr
    giz = dot(x1, a1z) + dot(x2, a2z) + bgz
    gin = dot(x1, a1n) + dot(x2, a2n) + bgn
    ghr = dot(h, whr) + bhr
    ghz = dot(h, whz) + bhz
    ghn = dot(h, whn) + bhn
    r = jax.nn.sigmoid(gir + ghr)
    z = jax.nn.sigmoid(giz + ghz)
    n = jnp.tanh(gin + r * ghn)
    return (1.0 - z) * n + z * h


def _phase_b_body(x1_ref, x2_ref, h_ref, *rest):
    (a1r, a1z, a1n, a2r, a2z, a2n, bgr, bgz, bgn,
     whr, whz, whn, bhr, bhz, bhn, wlin, blin, o_ref) = rest
    hn = _gru_from_parts(
        x1_ref[...], x2_ref[...], h_ref[...],
        (a1r[...], a1z[...], a1n[...], a2r[...], a2z[...], a2n[...],
         bgr[...], bgz[...], bgn[...], whr[...], whz[...], whn[...],
         bhr[...], bhz[...], bhn[...]))
    o_ref[...] = jnp.dot(hn, wlin[...], preferred_element_type=jnp.float32) + blin[...]


def _phase_b(x1, x2, h, parts, wlin_t, blin):
    grid = N // R
    mat = pl.BlockSpec((H, H), lambda i: (0, 0))
    vec = pl.BlockSpec((1, H), lambda i: (0, 0))
    row = pl.BlockSpec((R, H), lambda i: (i, 0))
    return pl.pallas_call(
        _phase_b_body,
        grid=(grid,),
        in_specs=[row, row, row] + [mat] * 6 + [vec] * 3 + [mat] * 3 + [vec] * 3
        + [mat, vec],
        out_specs=row,
        out_shape=jax.ShapeDtypeStruct((N, H), jnp.float32),
    )(x1, x2, h, *parts, wlin_t, blin)


def _phase_c_body(x1_ref, x2_ref, h_ref, *rest):
    (a1r, a1z, a1n, a2r, a2z, a2n, bgr, bgz, bgn,
     whr, whz, whn, bhr, bhz, bhn, wout, bout, o_ref) = rest
    hn = _gru_from_parts(
        x1_ref[...], x2_ref[...], h_ref[...],
        (a1r[...], a1z[...], a1n[...], a2r[...], a2z[...], a2n[...],
         bgr[...], bgz[...], bgn[...], whr[...], whz[...], whn[...],
         bhr[...], bhz[...], bhn[...]))
    logits = jnp.dot(hn, wout[...], preferred_element_type=jnp.float32) + bout[...]
    m = jnp.max(logits, axis=-1, keepdims=True)
    s = jnp.sum(jnp.exp(logits - m), axis=-1, keepdims=True)
    o_ref[...] = logits - m - jnp.log(s)


def _phase_c(x1, x2, h, parts, wout_t, bout):
    grid = N // R
    mat = pl.BlockSpec((H, H), lambda i: (0, 0))
    vec = pl.BlockSpec((1, H), lambda i: (0, 0))
    row = pl.BlockSpec((R, H), lambda i: (i, 0))
    return pl.pallas_call(
        _phase_c_body,
        grid=(grid,),
        in_specs=[row, row, row] + [mat] * 6 + [vec] * 3 + [mat] * 3 + [vec] * 3
        + [pl.BlockSpec((H, C), lambda i: (0, 0)), pl.BlockSpec((1, C), lambda i: (0, 0))],
        out_specs=pl.BlockSpec((R, C), lambda i: (i, 0)),
        out_shape=jax.ShapeDtypeStruct((N, C), jnp.float32),
    )(x1, x2, h, *parts, wout_t, bout)


def _fold_gru_parts(W_con, b_con, W_ih, b_ih, W_hh, b_hh):
    a1 = W_con[:, :H].T @ W_ih.T
    a2 = W_con[:, H:].T @ W_ih.T
    bg = (b_con @ W_ih.T + b_ih)[None]
    wh = W_hh.T
    bh = b_hh[None]
    sl = lambda m, k: m[:, k * H:(k + 1) * H]
    vl = lambda v, k: v[:, k * H:(k + 1) * H]
    return (sl(a1, 0), sl(a1, 1), sl(a1, 2),
            sl(a2, 0), sl(a2, 1), sl(a2, 2),
            vl(bg, 0), vl(bg, 1), vl(bg, 2),
            sl(wh, 0), sl(wh, 1), sl(wh, 2),
            vl(bh, 0), vl(bh, 1), vl(bh, 2))


SUB = 128
CHUNK_ROWS = 2
CHUNK_E = SUB * CHUNK_ROWS
BLK_CHUNKS = 8
BLK_ROWS = BLK_CHUNKS * CHUNK_ROWS
N_TILES = 16
EPAD = 1638400
ROWS2D = EPAD // SUB
EDGES_PER_TILE = EPAD // N_TILES
ROWS_PER_TILE = EDGES_PER_TILE // SUB
N_CHUNKS = ROWS_PER_TILE // CHUNK_ROWS
N_BLKS = N_CHUNKS // BLK_CHUNKS
NPAD = 50048
NODES_PER_TILE = NPAD // N_TILES


def _conv_pair(y, src2d, dst2d, w_all):
    mesh = plsc.VectorSubcoreMesh(core_axis_name="c", subcore_axis_name="s")

    @functools.partial(
        pl.kernel,
        mesh=mesh,
        compiler_params=pltpu.CompilerParams(use_tc_tiling_on_sc=False),
        out_type=jax.ShapeDtypeStruct((2, NPAD, H), jnp.float32),
        scratch_types=[
            pltpu.VMEM((BLK_ROWS, SUB), jnp.int32),
            pltpu.VMEM((BLK_ROWS, SUB), jnp.int32),
            pltpu.VMEM((BLK_ROWS, SUB), jnp.int32),
            pltpu.VMEM((BLK_ROWS, SUB), jnp.int32),
            pltpu.VMEM((BLK_ROWS, SUB), jnp.float32),
            pltpu.VMEM((BLK_ROWS, SUB), jnp.float32),
            pltpu.VMEM((CHUNK_E, H), jnp.float32),
            pltpu.VMEM((CHUNK_E, H), jnp.float32),
            pltpu.VMEM_SHARED((NPAD, H), jnp.float32),
            pltpu.SemaphoreType.DMA,
            pltpu.SemaphoreType.DMA,
            pltpu.SemaphoreType.DMA,
            pltpu.SemaphoreType.DMA,
            pltpu.SemaphoreType.DMA,
            pltpu.SemaphoreType.DMA,
        ],
    )
    def k(y_hbm, src_hbm, dst_hbm, w_hbm, out_hbm,
          sidx0, sidx1, didx0, didx1, wv0, wv1, rows0, rows1, acc,
          semg0, semg1, sems0, sems1, semi0, semi1):
        c = lax.axis_index("c")
        s = lax.axis_index("s")
        sidx = (sidx0, sidx1)
        didx = (didx0, didx1)
        w_v = (wv0, wv1)
        rows = (rows0, rows1)
        sem_g = (semg0, semg1)
        sem_s = (sems0, sems1)
        sem_i = (semi0, semi1)

        z = jnp.zeros((16,), jnp.float32)

        def zbody(r, carry):
            rows0[r, pl.ds(0, 16)] = z
            rows0[r, pl.ds(16, 16)] = z
            return carry

        lax.fori_loop(0, CHUNK_E, zbody, 0)
        base = s * NODES_PER_TILE
        for q in range(NODES_PER_TILE // CHUNK_E):
            pltpu.sync_copy(rows0, acc.at[pl.ds(base + q * CHUNK_E, CHUNK_E)])
        rem = NODES_PER_TILE % CHUNK_E
        if rem:
            pltpu.sync_copy(rows0.at[pl.ds(0, rem)],
                            acc.at[pl.ds(base + NODES_PER_TILE - rem, rem)])
        plsc.subcore_barrier()

        def issue_idx_block(blk, q):
            r0 = pl.multiple_of(s * ROWS_PER_TILE + blk * BLK_ROWS, BLK_ROWS)
            pltpu.async_copy(src_hbm.at[c, pl.ds(r0, BLK_ROWS)], sidx[q], sem_i[q])
            pltpu.async_copy(dst_hbm.at[c, pl.ds(r0, BLK_ROWS)], didx[q], sem_i[q])
            pltpu.async_copy(w_hbm.at[c, pl.ds(r0, BLK_ROWS)], w_v[q], sem_i[q])

        def drain_idx_block(q):
            pltpu.make_async_copy(src_hbm.at[c, pl.ds(0, BLK_ROWS)], sidx[q],
                                  sem_i[q]).wait()
            pltpu.make_async_copy(dst_hbm.at[c, pl.ds(0, BLK_ROWS)], didx[q],
                                  sem_i[q]).wait()
            pltpu.make_async_copy(w_hbm.at[c, pl.ds(0, BLK_ROWS)], w_v[q],
                                  sem_i[q]).wait()

        def fire_gathers(q, row0, b):
            for j in range(CHUNK_ROWS):
                pltpu.async_copy(y_hbm.at[sidx[q].at[row0 + j]],
                                 rows[b].at[pl.ds(j * SUB, SUB)], sem_g[b])

        def drain_gathers(b):
            pltpu.make_async_copy(y_hbm.at[pl.ds(0, CHUNK_E)], rows[b],
                                  sem_g[b]).wait()

        def fire_scatters(q, row0, b):
            for j in range(CHUNK_ROWS):
                pltpu.async_copy(rows[b].at[pl.ds(j * SUB, SUB)],
                                 acc.at[didx[q].at[row0 + j]], sem_s[b],
                                 add=True)

        def drain_scatters(b):
            pltpu.make_async_copy(y_hbm.at[pl.ds(0, CHUNK_E)], rows[b],
                                  sem_s[b]).wait()

        def scale_chunk(b, q, crow):
            rb = rows[b]

            @plsc.parallel_loop(0, CHUNK_E // 16, unroll=2)
            def _scale(t):
                w16 = w_v[q][crow + (t >> 3), pl.ds((t & 7) * 16, 16)]
                for l in range(16):
                    lv = jnp.full((16,), l, jnp.int32)
                    ws = w16.at[lv].get(mode="promise_in_bounds")
                    e = t * 16 + l
                    rb[e, pl.ds(0, 16)] = rb[e, pl.ds(0, 16)] * ws
                    rb[e, pl.ds(16, 16)] = rb[e, pl.ds(16, 16)] * ws

        issue_idx_block(0, 0)
        drain_idx_block(0)
        fire_gathers(0, 0, 0)

        def blk_pair_body(pb, carry):
            for q in range(2):
                sb = 2 * pb + q
                q1 = 1 - q
                for cidx in range(BLK_CHUNKS):
                    b = cidx % 2
                    b1 = 1 - b

                    if cidx == 0:
                        @pl.when(sb > 0)
                        def _drain_prev():
                            drain_scatters(b1)

                        @pl.when(sb < N_BLKS - 1)
                        def _refill():
                            issue_idx_block(sb + 1, q1)
                    else:
                        drain_scatters(b1)

                    if cidx < BLK_CHUNKS - 1:
                        fire_gathers(q, (cidx + 1) * CHUNK_ROWS, b1)
                    else:
                        @pl.when(sb < N_BLKS - 1)
                        def _next_blk_gather():
                            drain_idx_block(q1)
                            fire_gathers(q1, 0, b1)

                    drain_gathers(b)
                    scale_chunk(b, q, cidx * CHUNK_ROWS)
                    fire_scatters(q, cidx * CHUNK_ROWS, b)
            return carry

        lax.fori_loop(0, N_BLKS // 2, blk_pair_body, 0)
        drain_scatters(1)
        plsc.subcore_barrier()
        pltpu.sync_copy(acc.at[pl.ds(base, NODES_PER_TILE)],
                        out_hbm.at[c, pl.ds(base, NODES_PER_TILE)])

    out = k(y, src2d, dst2d, w_all)
    return out[0], out[1]


def kernel(x, edge_index, edge_weight, edge_index_re, edge_weight_re,
           W_first, b_first, W_con1, b_con1, W_con2, b_con2,
           W_lin1, b_lin1, W_out, b_out, W_ih, W_hh, b_ih, b_hh):
    parts1 = _fold_gru_parts(W_con1, b_con1, W_ih, b_ih, W_hh, b_hh)
    parts2 = _fold_gru_parts(W_con2, b_con2, W_ih, b_ih, W_hh, b_hh)

    pad_i = jnp.zeros((EPAD - E,), jnp.int32)
    pad_f = jnp.zeros((EPAD - E,), jnp.float32)
    cat_i = lambda a: jnp.concatenate([a, pad_i]).reshape(ROWS2D, SUB)
    cat_f = lambda a: jnp.concatenate([a, pad_f]).reshape(ROWS2D, SUB)
    src2d = jnp.stack([cat_i(edge_index[0]), cat_i(edge_index_re[0])])
    dst2d = jnp.stack([cat_i(edge_index[1]), cat_i(edge_index_re[1])])
    w_all = jnp.stack([cat_f(edge_weight), cat_f(edge_weight_re)])

    y = _phase_a(x, W_first.T, b_first[None])
    x1, x2 = _conv_pair(y, src2d, dst2d, w_all)
    xm = _phase_b(x1, x2, y, parts1, W_lin1.T, b_lin1[None])
    x1b, x2b = _conv_pair(xm, src2d, dst2d, w_all)
    return _phase_c(x1b, x2b, y, parts2, W_out.T, b_out[None])
